# Initial kernel scaffold; baseline (speedup 1.0000x reference)
import jax
import jax.numpy as jnp
from jax.experimental import pallas as pl


def kernel(x_lidar, x_radar1, x_radar2, ei_lidar_lidar, ei_radar1_lidar, ei_radar2_lidar, ei_lidar_radar1, ei_lidar_radar2, params):
    raise NotImplementedError("write your pallas kernel here")


# speedup vs baseline: 1.0003x; 1.0003x over previous
import functools

import jax
import jax.numpy as jnp
from jax.experimental import pallas as pl
from jax.experimental.pallas import tpu as pltpu

Nl, Nr1, Nr2 = 50000, 20000, 20000
HID, H, C = 128, 4, 32
NODE_N = {"lidar": Nl, "radar1": Nr1, "radar2": Nr2}
EDGE_DEFS = [
    ("lidar", "lidar", "ei_lidar_lidar", True),
    ("radar1", "lidar", "ei_radar1_lidar", False),
    ("radar2", "lidar", "ei_radar2_lidar", False),
    ("lidar", "radar1", "ei_lidar_radar1", False),
    ("lidar", "radar2", "ei_lidar_radar2", False),
]

_BLK = 400


def _enc_body(x_ref, w_ref, b_ref, g_ref, beta_ref, o_ref):
    h = jnp.dot(x_ref[...], w_ref[...].T, preferred_element_type=jnp.float32)
    h = h + b_ref[...]
    h = jnp.where(h > 0, h, 0.01 * h)
    mu = jnp.mean(h, axis=-1, keepdims=True)
    var = jnp.mean((h - mu) ** 2, axis=-1, keepdims=True)
    o_ref[...] = (h - mu) * jax.lax.rsqrt(var + 1e-5) * g_ref[...] + beta_ref[...]


def _encode(x, pe):
    n = x.shape[0]
    grid = n // _BLK
    return pl.pallas_call(
        _enc_body,
        grid=(grid,),
        in_specs=[
            pl.BlockSpec((_BLK, HID), lambda i: (i, 0)),
            pl.BlockSpec((HID, HID), lambda i: (0, 0)),
            pl.BlockSpec((1, HID), lambda i: (0, 0)),
            pl.BlockSpec((1, HID), lambda i: (0, 0)),
            pl.BlockSpec((1, HID), lambda i: (0, 0)),
        ],
        out_specs=pl.BlockSpec((_BLK, HID), lambda i: (i, 0)),
        out_shape=jax.ShapeDtypeStruct((n, HID), jnp.float32),
    )(x, pe["W"], pe["b"][None, :], pe["g"][None, :], pe["beta"][None, :])


def _gat(x_src, x_dst, src, dst, p, num_dst, add_self_loops):
    hs = (x_src @ p["W"].T).reshape(-1, H, C)
    hd = (x_dst @ p["W"].T).reshape(-1, H, C)
    if add_self_loops:
        loop = jnp.arange(num_dst, dtype=src.dtype)
        src = jnp.concatenate([src, loop])
        dst = jnp.concatenate([dst, loop])
    a_s = jnp.sum(hs * p["a_src"], axis=-1)
    a_d = jnp.sum(hd * p["a_dst"], axis=-1)
    alpha = jax.nn.leaky_relu(a_s[src] + a_d[dst], 0.2)
    amax = jax.ops.segment_max(alpha, dst, num_segments=num_dst)
    amax = jnp.where(jnp.isfinite(amax), amax, 0.0)
    ex = jnp.exp(alpha - amax[dst])
    den = jax.ops.segment_sum(ex, dst, num_segments=num_dst)
    coef = ex / (den[dst] + 1e-16)
    out = jax.ops.segment_sum(hs[src] * coef[..., None], dst, num_segments=num_dst)
    return out.reshape(num_dst, H * C) + p["bias"]


def kernel(x_lidar, x_radar1, x_radar2, ei_lidar_lidar, ei_radar1_lidar, ei_radar2_lidar, ei_lidar_radar1, ei_lidar_radar2, params):
    ei = {
        "ei_lidar_lidar": ei_lidar_lidar,
        "ei_radar1_lidar": ei_radar1_lidar,
        "ei_radar2_lidar": ei_radar2_lidar,
        "ei_lidar_radar1": ei_lidar_radar1,
        "ei_lidar_radar2": ei_lidar_radar2,
    }
    xs = {"lidar": x_lidar, "radar1": x_radar1, "radar2": x_radar2}
    out = {}
    for nt, x in xs.items():
        out[nt] = _encode(x, params["enc"][nt])
    for lp in params["convs"]:
        agg = {nt: jnp.zeros((NODE_N[nt], HID), jnp.float32) for nt in NODE_N}
        for (s, d, name, sl) in EDGE_DEFS:
            e = ei[name]
            agg[d] = agg[d] + _gat(out[s], out[d], e[0], e[1], lp[s + "__" + d], NODE_N[d], sl)
        out = {nt: jax.nn.leaky_relu(v, 0.01) for nt, v in agg.items()}
    lh = params["l_head"]
    res_l = jax.nn.leaky_relu(out["lidar"] @ lh["W1"].T + lh["b1"], 0.01) @ lh["W2"].T + lh["b2"]
    mu_l = res_l[:, :2]
    sigma_l = jax.nn.softplus(res_l[:, 2:3]) + 1e-4
    rh = params["r_head"]

    def rad(nt):
        r = jax.nn.leaky_relu(out[nt] @ rh["W1"].T + rh["b1"], 0.01) @ rh["W2"].T + rh["b2"]
        return r[:, 0:1], jax.nn.softplus(r[:, 1:2]) + 1e-4

    mu_r1, sigma_r1 = rad("radar1")
    mu_r2, sigma_r2 = rad("radar2")
    return (mu_l, sigma_l, mu_r1, sigma_r1, mu_r2, sigma_r2)


# speedup vs baseline: 22.1408x; 22.1332x over previous
import functools

import jax
import jax.numpy as jnp
from jax import lax
from jax.experimental import pallas as pl
from jax.experimental.pallas import tpu as pltpu
from jax.experimental.pallas import tpu_sc as plsc

Nl, Nr1, Nr2 = 50000, 20000, 20000
HID, H, C = 128, 4, 32
L = 16
NW = 32
ROWS = 392
TSW = 144
EB = 128

SRC_N = [Nl, Nr1, Nr2, Nl, Nl]
DST_N = [Nl, Nl, Nl, Nr1, Nr2]
E_T = [300000, 80000, 80000, 80000, 80000]
CH_T = [-(-n // ROWS) for n in DST_N]
NCHUNK = sum(CH_T)
DPAD_T = [c * ROWS for c in CH_T]
DST_OFF = [sum(DPAD_T[:i]) for i in range(5)]
ND_CAT = sum(DPAD_T)
SRC_OFF = [sum(SRC_N[:i]) for i in range(5)]
NS_CAT = sum(SRC_N)
E_OFF = [sum(E_T[:i]) for i in range(5)]
E_CAT = sum(E_T)
MAXCH = -(-NCHUNK // NW)

_BLK = 400


def _enc_body(x_ref, w_ref, b_ref, g_ref, beta_ref, o_ref):
    h = jnp.dot(x_ref[...], w_ref[...], preferred_element_type=jnp.float32)
    h = h + b_ref[...]
    h = jnp.where(h > 0, h, 0.01 * h)
    mu = jnp.mean(h, axis=-1, keepdims=True)
    var = jnp.mean((h - mu) ** 2, axis=-1, keepdims=True)
    o_ref[...] = (h - mu) * jax.lax.rsqrt(var + 1e-5) * g_ref[...] + beta_ref[...]


def _encode(x, pe):
    n = x.shape[0]
    return pl.pallas_call(
        _enc_body,
        grid=(n // _BLK,),
        in_specs=[
            pl.BlockSpec((_BLK, HID), lambda i: (i, 0)),
            pl.BlockSpec((HID, HID), lambda i: (0, 0)),
            pl.BlockSpec((1, HID), lambda i: (0, 0)),
            pl.BlockSpec((1, HID), lambda i: (0, 0)),
            pl.BlockSpec((1, HID), lambda i: (0, 0)),
        ],
        out_specs=pl.BlockSpec((_BLK, HID), lambda i: (i, 0)),
        out_shape=jax.ShapeDtypeStruct((n, HID), jnp.float32),
    )(x, pe["W"].T, pe["b"][None, :], pe["g"][None, :], pe["beta"][None, :])


def _mm_body(x_ref, w_ref, o_ref):
    o_ref[...] = jnp.dot(x_ref[...], w_ref[...], preferred_element_type=jnp.float32)


def _matmul(x, w):
    n, k = x.shape[0], w.shape[1]
    return pl.pallas_call(
        _mm_body,
        grid=(n // _BLK,),
        in_specs=[
            pl.BlockSpec((_BLK, HID), lambda i: (i, 0)),
            pl.BlockSpec((HID, k), lambda i: (0, 0)),
        ],
        out_specs=pl.BlockSpec((_BLK, k), lambda i: (i, 0)),
        out_shape=jax.ShapeDtypeStruct((n, k), jnp.float32),
    )(x, w)


def _sc_body(srcs_hbm, dsts_hbm, ts_hbm, td_hbm, desc_hbm, out_hbm,
             desc_v, src_v, dst_v, rows_v, td_v, num_v, sem):
    wid = lax.axis_index("s") * 2 + lax.axis_index("c")
    iota = lax.iota(jnp.int32, L)

    def lane(vec, j):
        return jnp.sum(jnp.where(iota == j, vec, 0.0)).astype(jnp.int32)

    def chunk_body(ci, carry):
        chunk = wid + ci * NW

        @pl.when(chunk < NCHUNK)
        def _():
            pltpu.sync_copy(desc_hbm.at[pl.ds(chunk * 16, 16)], desc_v)
            dvec = desc_v[...]
            e0 = lane(dvec, 0)
            e1 = lane(dvec, 1)
            dbase = lane(dvec, 2)
            pltpu.sync_copy(td_hbm.at[pl.ds(dbase * 8, ROWS * 8)], td_v)

            def zbody(j, c):
                num_v[pl.ds(j * L, L)] = jnp.zeros((L,), jnp.float32)
                return c

            lax.fori_loop(0, ROWS * TSW // L, zbody, 0)

            eA = (e0 // 8) * 8
            nblk = (e1 - eA + EB - 1) // EB

            def blk_body(b, c):
                ebase = eA + b * EB
                pltpu.sync_copy(srcs_hbm.at[pl.ds(ebase, EB)], src_v)
                pltpu.sync_copy(dsts_hbm.at[pl.ds(ebase, EB)], dst_v)
                pltpu.async_copy(ts_hbm.at[src_v], rows_v, sem).wait()

                def grp_body(g, cc):
                    eloc = g * L + iota
                    eabs = ebase + eloc
                    valid = (eabs >= e0) & (eabs < e1)
                    dstg = dst_v[pl.ds(g * L, L)]
                    ldr = jnp.where(valid, dstg - dbase, 0)
                    exs = []
                    for h in range(H):
                        a_s = plsc.load_gather(
                            rows_v, [eloc, jnp.full((L,), 128 + h, jnp.int32)])
                        a_d = plsc.load_gather(td_v, [ldr * 8 + h])
                        tt = plsc.load_gather(td_v, [ldr * 8 + 4 + h])
                        s = a_s + a_d
                        alpha = jnp.maximum(s, 0.2 * s)
                        exs.append(jnp.where(valid, jnp.exp(alpha - tt), 0.0))
                    vi = valid.astype(jnp.int32)

                    def edge_body(e, c2):
                        esel = jnp.full((L,), 0, jnp.int32) + e
                        exsp = [exs[h].at[esel].get(mode="promise_in_bounds")
                                for h in range(H)]
                        ldsp = ldr.at[esel].get(mode="promise_in_bounds")
                        msk = vi.at[esel].get(mode="promise_in_bounds") > 0
                        rowbase = ldsp * TSW
                        erow = g * L + e
                        for k in range(8):
                            hv = rows_v[erow, pl.ds(k * L, L)]
                            val = hv * exsp[k // 2]
                            plsc.addupdate_scatter(
                                num_v, [rowbase + (k * L) + iota], val, mask=msk)
                        denv = jnp.where(
                            iota == 0, exsp[0],
                            jnp.where(iota == 1, exsp[1],
                                      jnp.where(iota == 2, exsp[2],
                                                jnp.where(iota == 3, exsp[3], 0.0))))
                        plsc.addupdate_scatter(
                            num_v, [rowbase + 128 + iota], denv, mask=msk)
                        return c2

                    lax.fori_loop(0, L, edge_body, 0)
                    return cc

                lax.fori_loop(0, EB // L, grp_body, 0)
                return c

            lax.fori_loop(0, nblk, blk_body, 0)
            pltpu.sync_copy(num_v, out_hbm.at[pl.ds(dbase * TSW, ROWS * TSW)])

        return carry

    lax.fori_loop(0, MAXCH, chunk_body, 0)


_sc_edge_kernel = functools.partial(
    pl.kernel,
    out_type=jax.ShapeDtypeStruct((ND_CAT * TSW,), jnp.float32),
    mesh=plsc.VectorSubcoreMesh(core_axis_name="c", subcore_axis_name="s",
                                num_cores=2, num_subcores=16),
    scratch_types=[
        pltpu.VMEM((16,), jnp.float32),
        pltpu.VMEM((EB,), jnp.int32),
        pltpu.VMEM((EB,), jnp.int32),
        pltpu.VMEM((EB, TSW), jnp.float32),
        pltpu.VMEM((ROWS * 8,), jnp.float32),
        pltpu.VMEM((ROWS * TSW,), jnp.float32),
        pltpu.SemaphoreType.DMA,
    ],
    compiler_params=pltpu.CompilerParams(needs_layout_passes=False, use_tc_tiling_on_sc=False),
)(_sc_body)


def _leaky(x, s):
    return jnp.where(x > 0, x, s * x)


def _prep_edges(eis):
    srcs_all, dsts_all, descs = [], [], []
    for t, ei in enumerate(eis):
        src, dst = ei[0], ei[1]
        perm = jnp.argsort(dst)
        s = src[perm] + SRC_OFF[t]
        d = dst[perm]
        bnd = jnp.searchsorted(d, jnp.arange(CH_T[t] + 1, dtype=jnp.int32) * ROWS).astype(jnp.int32)
        e0 = bnd[:-1] + E_OFF[t]
        e1 = bnd[1:] + E_OFF[t]
        dbase = (DST_OFF[t] + jnp.arange(CH_T[t], dtype=jnp.int32) * ROWS)
        z = jnp.zeros_like(e0)
        descs.append(jnp.stack([e0, e1, dbase] + [z] * 13, axis=1))
        srcs_all.append(s)
        dsts_all.append(d + DST_OFF[t])
    pad = jnp.zeros((EB,), jnp.int32)
    srcs = jnp.concatenate(srcs_all + [pad])
    dsts = jnp.concatenate(dsts_all + [pad])
    desc = jnp.concatenate(descs).astype(jnp.float32).reshape(-1)
    return srcs, dsts, desc


def _wcat(p):
    Wt = p["W"].T
    ws = jnp.einsum("ihc,hc->ih", Wt.reshape(HID, H, C), p["a_src"][0])
    wd = jnp.einsum("ihc,hc->ih", Wt.reshape(HID, H, C), p["a_dst"][0])
    return jnp.concatenate([Wt, ws, wd, jnp.zeros((HID, 8))], axis=1)


def _layer(xs, lp, srcs, dsts, desc):
    p_ll, p_r1l, p_r2l, p_lr1, p_lr2 = (lp["lidar__lidar"], lp["radar1__lidar"],
                                        lp["radar2__lidar"], lp["lidar__radar1"],
                                        lp["lidar__radar2"])
    wl = jnp.concatenate([_wcat(p_ll), _wcat(p_lr1), _wcat(p_lr2),
                          jnp.einsum("ihc,hc->ih", p_r1l["W"].T.reshape(HID, H, C), p_r1l["a_dst"][0]),
                          jnp.einsum("ihc,hc->ih", p_r2l["W"].T.reshape(HID, H, C), p_r2l["a_dst"][0])],
                         axis=1)
    big_l = _matmul(xs["lidar"], jnp.concatenate([wl, jnp.zeros((HID, 8))], axis=1))
    wr1 = jnp.concatenate([_wcat(p_r1l),
                           jnp.einsum("ihc,hc->ih", p_lr1["W"].T.reshape(HID, H, C), p_lr1["a_dst"][0]),
                           jnp.zeros((HID, 4))], axis=1)
    big_r1 = _matmul(xs["radar1"], wr1)
    wr2 = jnp.concatenate([_wcat(p_r2l),
                           jnp.einsum("ihc,hc->ih", p_lr2["W"].T.reshape(HID, H, C), p_lr2["a_dst"][0]),
                           jnp.zeros((HID, 4))], axis=1)
    big_r2 = _matmul(xs["radar2"], wr2)

    ts_t = [big_l[:, 0:144], big_r1[:, 0:144], big_r2[:, 0:144],
            big_l[:, 144:288], big_l[:, 288:432]]
    ad_t = [big_l[:, 132:136], big_l[:, 432:436], big_l[:, 436:440],
            big_r1[:, 144:148], big_r2[:, 144:148]]

    td_parts, t_ll = [], None
    for t in range(5):
        a_s = ts_t[t][:, 128:132]
        maxs = jnp.max(a_s, axis=0)
        tvals = _leaky(ad_t[t] + maxs, 0.2)
        if t == 0:
            t_ll = tvals
        td = jnp.concatenate([ad_t[t], tvals], axis=1)
        td = jnp.concatenate(
            [td, jnp.zeros((DPAD_T[t] - DST_N[t], 8), jnp.float32)], axis=0)
        td_parts.append(td)
    ts_cat = jnp.concatenate(ts_t, axis=0)
    td_cat = jnp.concatenate(td_parts, axis=0).reshape(-1)

    out_flat = _sc_edge_kernel(srcs, dsts, ts_cat, td_cat, desc)
    out2 = out_flat.reshape(ND_CAT, TSW)

    agg = {"lidar": 0.0, "radar1": 0.0, "radar2": 0.0}
    dst_types = ["lidar", "lidar", "lidar", "radar1", "radar2"]
    biases = [p_ll["bias"], p_r1l["bias"], p_r2l["bias"], p_lr1["bias"], p_lr2["bias"]]
    for t in range(5):
        nd = DST_N[t]
        seg = out2[DST_OFF[t]:DST_OFF[t] + nd]
        num = seg[:, :128]
        den = seg[:, 128:132]
        if t == 0:
            a_sl = ts_t[0][:, 128:132]
            a_dl = ts_t[0][:, 132:136]
            ex_self = jnp.exp(_leaky(a_sl + a_dl, 0.2) - t_ll)
            den = den + ex_self
            num = num + jnp.repeat(ex_self, C, axis=1) * ts_t[0][:, :128]
        outt = num / (jnp.repeat(den, C, axis=1) + 1e-16) + biases[t]
        agg[dst_types[t]] = agg[dst_types[t]] + outt
    return {nt: _leaky(v, 0.01) for nt, v in agg.items()}


def kernel(x_lidar, x_radar1, x_radar2, ei_lidar_lidar, ei_radar1_lidar,
           ei_radar2_lidar, ei_lidar_radar1, ei_lidar_radar2, params):
    srcs, dsts, desc = _prep_edges([ei_lidar_lidar, ei_radar1_lidar,
                                    ei_radar2_lidar, ei_lidar_radar1,
                                    ei_lidar_radar2])
    xs = {"lidar": _encode(x_lidar, params["enc"]["lidar"]),
          "radar1": _encode(x_radar1, params["enc"]["radar1"]),
          "radar2": _encode(x_radar2, params["enc"]["radar2"])}
    for lp in params["convs"]:
        xs = _layer(xs, lp, srcs, dsts, desc)

    lh = params["l_head"]
    res_l = _leaky(xs["lidar"] @ lh["W1"].T + lh["b1"], 0.01) @ lh["W2"].T + lh["b2"]
    mu_l = res_l[:, :2]
    sigma_l = jax.nn.softplus(res_l[:, 2:3]) + 1e-4
    rh = params["r_head"]

    def rad(nt):
        r = _leaky(xs[nt] @ rh["W1"].T + rh["b1"], 0.01) @ rh["W2"].T + rh["b2"]
        return r[:, 0:1], jax.nn.softplus(r[:, 1:2]) + 1e-4

    mu_r1, sigma_r1 = rad("radar1")
    mu_r2, sigma_r2 = rad("radar2")
    return (mu_l, sigma_l, mu_r1, sigma_r1, mu_r2, sigma_r2)


# speedup vs baseline: 24.7223x; 1.1166x over previous
import functools

import jax
import jax.numpy as jnp
from jax import lax
from jax.experimental import pallas as pl
from jax.experimental.pallas import tpu as pltpu
from jax.experimental.pallas import tpu_sc as plsc

Nl, Nr1, Nr2 = 50000, 20000, 20000
HID, H, C = 128, 4, 32
L = 16
NW = 32
ROWS = 392
TSW = 144
EB = 128

SRC_N = [Nl, Nr1, Nr2, Nl, Nl]
DST_N = [Nl, Nl, Nl, Nr1, Nr2]
E_T = [300000, 80000, 80000, 80000, 80000]
CH_T = [-(-n // ROWS) for n in DST_N]
NCHUNK = sum(CH_T)
DPAD_T = [c * ROWS for c in CH_T]
DST_OFF = [sum(DPAD_T[:i]) for i in range(5)]
ND_CAT = sum(DPAD_T)
SRC_OFF = [sum(SRC_N[:i]) for i in range(5)]
NS_CAT = sum(SRC_N)
E_OFF = [sum(E_T[:i]) for i in range(5)]
E_CAT = sum(E_T)
MAXCH = -(-NCHUNK // NW)

_BLK = 400


def _enc_body(x_ref, w_ref, b_ref, g_ref, beta_ref, o_ref):
    h = jnp.dot(x_ref[...], w_ref[...], preferred_element_type=jnp.float32)
    h = h + b_ref[...]
    h = jnp.where(h > 0, h, 0.01 * h)
    mu = jnp.mean(h, axis=-1, keepdims=True)
    var = jnp.mean((h - mu) ** 2, axis=-1, keepdims=True)
    o_ref[...] = (h - mu) * jax.lax.rsqrt(var + 1e-5) * g_ref[...] + beta_ref[...]


def _encode(x, pe):
    n = x.shape[0]
    return pl.pallas_call(
        _enc_body,
        grid=(n // _BLK,),
        in_specs=[
            pl.BlockSpec((_BLK, HID), lambda i: (i, 0)),
            pl.BlockSpec((HID, HID), lambda i: (0, 0)),
            pl.BlockSpec((1, HID), lambda i: (0, 0)),
            pl.BlockSpec((1, HID), lambda i: (0, 0)),
            pl.BlockSpec((1, HID), lambda i: (0, 0)),
        ],
        out_specs=pl.BlockSpec((_BLK, HID), lambda i: (i, 0)),
        out_shape=jax.ShapeDtypeStruct((n, HID), jnp.float32),
    )(x, pe["W"].T, pe["b"][None, :], pe["g"][None, :], pe["beta"][None, :])


def _mm_body(x_ref, w_ref, o_ref):
    o_ref[...] = jnp.dot(x_ref[...], w_ref[...], preferred_element_type=jnp.float32)


def _matmul(x, w):
    n, k = x.shape[0], w.shape[1]
    return pl.pallas_call(
        _mm_body,
        grid=(n // _BLK,),
        in_specs=[
            pl.BlockSpec((_BLK, HID), lambda i: (i, 0)),
            pl.BlockSpec((HID, k), lambda i: (0, 0)),
        ],
        out_specs=pl.BlockSpec((_BLK, k), lambda i: (i, 0)),
        out_shape=jax.ShapeDtypeStruct((n, k), jnp.float32),
    )(x, w)


def _sc_body(srcs_hbm, dsts_hbm, ts_hbm, td_hbm, desc_hbm, out_hbm,
             desc_v, src_v, dst_v, rows_v, td_v, num_v, sem0, sem1):
    wid = lax.axis_index("s") * 2 + lax.axis_index("c")
    iota = lax.iota(jnp.int32, L)
    sems = [sem0, sem1]

    def lane(vec, j):
        return jnp.sum(jnp.where(iota == j, vec, 0.0)).astype(jnp.int32)

    def chunk_body(ci, carry):
        chunk = wid + ci * NW

        @pl.when(chunk < NCHUNK)
        def _():
            pltpu.sync_copy(desc_hbm.at[pl.ds(chunk * 16, 16)], desc_v)
            dvec = desc_v[...]
            e0 = lane(dvec, 0)
            e1 = lane(dvec, 1)
            dbase = lane(dvec, 2)
            pltpu.sync_copy(td_hbm.at[pl.ds(dbase * 8, ROWS * 8)], td_v)

            def zbody(j, c):
                for u in range(4):
                    num_v[pl.ds((j * 4 + u) * L, L)] = jnp.zeros((L,), jnp.float32)
                return c

            lax.fori_loop(0, ROWS * TSW // L // 4, zbody, 0)

            eA = (e0 // 8) * 8
            nblk = (e1 - eA + EB - 1) // EB

            def prefetch(bb, slot):
                ebb = eA + bb * EB
                pltpu.sync_copy(srcs_hbm.at[pl.ds(ebb, EB)], src_v.at[slot])
                pltpu.sync_copy(dsts_hbm.at[pl.ds(ebb, EB)], dst_v.at[slot])
                pltpu.async_copy(ts_hbm.at[src_v.at[slot]], rows_v.at[slot],
                                 sems[slot])

            def compute(b, slot):
                ebase = eA + b * EB
                rv = rows_v.at[slot]

                def grp_body(g, cc):
                    eloc = g * L + iota
                    eabs = ebase + eloc
                    valid = (eabs >= e0) & (eabs < e1)
                    dstg = dst_v[slot, pl.ds(g * L, L)]
                    ldr = jnp.where(valid, dstg - dbase, 0)
                    exs = []
                    for h in range(H):
                        a_s = plsc.load_gather(
                            rv, [eloc, jnp.full((L,), 128 + h, jnp.int32)])
                        a_d = plsc.load_gather(td_v, [ldr * 8 + h])
                        tt = plsc.load_gather(td_v, [ldr * 8 + 4 + h])
                        s = a_s + a_d
                        alpha = jnp.maximum(s, 0.2 * s)
                        exs.append(jnp.where(valid, jnp.exp(alpha - tt), 0.0))
                    vi = valid.astype(jnp.int32)

                    def edge_body(e, c2):
                        esel = jnp.full((L,), 0, jnp.int32) + e
                        exsp = [exs[h].at[esel].get(mode="promise_in_bounds")
                                for h in range(H)]
                        ldsp = ldr.at[esel].get(mode="promise_in_bounds")
                        msk = vi.at[esel].get(mode="promise_in_bounds") > 0
                        rowbase = ldsp * TSW
                        erow = g * L + e
                        for k in range(8):
                            hv = rows_v[slot, erow, pl.ds(k * L, L)]
                            val = hv * exsp[k // 2]
                            plsc.addupdate_scatter(
                                num_v, [rowbase + (k * L) + iota], val, mask=msk)
                        denv = jnp.where(
                            iota == 0, exsp[0],
                            jnp.where(iota == 1, exsp[1],
                                      jnp.where(iota == 2, exsp[2],
                                                jnp.where(iota == 3, exsp[3], 0.0))))
                        plsc.addupdate_scatter(
                            num_v, [rowbase + 128 + iota], denv, mask=msk)
                        return c2

                    lax.fori_loop(0, L, edge_body, 0)
                    return cc

                lax.fori_loop(0, EB // L, grp_body, 0)

            @pl.when(nblk > 0)
            def _():
                prefetch(0, 0)

            def pair_body(p, c):
                for j in range(2):
                    b = p * 2 + j

                    @pl.when(b < nblk)
                    def _():
                        @pl.when(b + 1 < nblk)
                        def _():
                            prefetch(b + 1, 1 - j)

                        pltpu.make_async_copy(
                            ts_hbm.at[src_v.at[j]], rows_v.at[j],
                            sems[j]).wait()
                        compute(b, j)

                return c

            lax.fori_loop(0, (nblk + 1) // 2, pair_body, 0)
            pltpu.sync_copy(num_v, out_hbm.at[pl.ds(dbase * TSW, ROWS * TSW)])

        return carry

    lax.fori_loop(0, MAXCH, chunk_body, 0)


_sc_edge_kernel = functools.partial(
    pl.kernel,
    out_type=jax.ShapeDtypeStruct((ND_CAT * TSW,), jnp.float32),
    mesh=plsc.VectorSubcoreMesh(core_axis_name="c", subcore_axis_name="s",
                                num_cores=2, num_subcores=16),
    scratch_types=[
        pltpu.VMEM((16,), jnp.float32),
        pltpu.VMEM((2, EB), jnp.int32),
        pltpu.VMEM((2, EB), jnp.int32),
        pltpu.VMEM((2, EB, TSW), jnp.float32),
        pltpu.VMEM((ROWS * 8,), jnp.float32),
        pltpu.VMEM((ROWS * TSW,), jnp.float32),
        pltpu.SemaphoreType.DMA,
        pltpu.SemaphoreType.DMA,
    ],
    compiler_params=pltpu.CompilerParams(needs_layout_passes=False, use_tc_tiling_on_sc=False),
)(_sc_body)


def _leaky(x, s):
    return jnp.where(x > 0, x, s * x)


def _prep_edges(eis):
    srcs_all, dsts_all, descs = [], [], []
    for t, ei in enumerate(eis):
        src, dst = ei[0], ei[1]
        perm = jnp.argsort(dst)
        s = src[perm] + SRC_OFF[t]
        d = dst[perm]
        bnd = jnp.searchsorted(d, jnp.arange(CH_T[t] + 1, dtype=jnp.int32) * ROWS).astype(jnp.int32)
        e0 = bnd[:-1] + E_OFF[t]
        e1 = bnd[1:] + E_OFF[t]
        dbase = (DST_OFF[t] + jnp.arange(CH_T[t], dtype=jnp.int32) * ROWS)
        z = jnp.zeros_like(e0)
        descs.append(jnp.stack([e0, e1, dbase] + [z] * 13, axis=1))
        srcs_all.append(s)
        dsts_all.append(d + DST_OFF[t])
    pad = jnp.zeros((EB,), jnp.int32)
    srcs = jnp.concatenate(srcs_all + [pad])
    dsts = jnp.concatenate(dsts_all + [pad])
    desc = jnp.concatenate(descs).astype(jnp.float32).reshape(-1)
    return srcs, dsts, desc


def _wcat(p):
    Wt = p["W"].T
    ws = jnp.einsum("ihc,hc->ih", Wt.reshape(HID, H, C), p["a_src"][0])
    wd = jnp.einsum("ihc,hc->ih", Wt.reshape(HID, H, C), p["a_dst"][0])
    return jnp.concatenate([Wt, ws, wd, jnp.zeros((HID, 8))], axis=1)


def _layer(xs, lp, srcs, dsts, desc):
    p_ll, p_r1l, p_r2l, p_lr1, p_lr2 = (lp["lidar__lidar"], lp["radar1__lidar"],
                                        lp["radar2__lidar"], lp["lidar__radar1"],
                                        lp["lidar__radar2"])
    wl = jnp.concatenate([_wcat(p_ll), _wcat(p_lr1), _wcat(p_lr2),
                          jnp.einsum("ihc,hc->ih", p_r1l["W"].T.reshape(HID, H, C), p_r1l["a_dst"][0]),
                          jnp.einsum("ihc,hc->ih", p_r2l["W"].T.reshape(HID, H, C), p_r2l["a_dst"][0])],
                         axis=1)
    big_l = _matmul(xs["lidar"], jnp.concatenate([wl, jnp.zeros((HID, 8))], axis=1))
    wr1 = jnp.concatenate([_wcat(p_r1l),
                           jnp.einsum("ihc,hc->ih", p_lr1["W"].T.reshape(HID, H, C), p_lr1["a_dst"][0]),
                           jnp.zeros((HID, 4))], axis=1)
    big_r1 = _matmul(xs["radar1"], wr1)
    wr2 = jnp.concatenate([_wcat(p_r2l),
                           jnp.einsum("ihc,hc->ih", p_lr2["W"].T.reshape(HID, H, C), p_lr2["a_dst"][0]),
                           jnp.zeros((HID, 4))], axis=1)
    big_r2 = _matmul(xs["radar2"], wr2)

    ts_t = [big_l[:, 0:144], big_r1[:, 0:144], big_r2[:, 0:144],
            big_l[:, 144:288], big_l[:, 288:432]]
    ad_t = [big_l[:, 132:136], big_l[:, 432:436], big_l[:, 436:440],
            big_r1[:, 144:148], big_r2[:, 144:148]]

    td_parts, t_ll = [], None
    for t in range(5):
        a_s = ts_t[t][:, 128:132]
        maxs = jnp.max(a_s, axis=0)
        tvals = _leaky(ad_t[t] + maxs, 0.2)
        if t == 0:
            t_ll = tvals
        td = jnp.concatenate([ad_t[t], tvals], axis=1)
        td = jnp.concatenate(
            [td, jnp.zeros((DPAD_T[t] - DST_N[t], 8), jnp.float32)], axis=0)
        td_parts.append(td)
    ts_cat = jnp.concatenate(ts_t, axis=0)
    td_cat = jnp.concatenate(td_parts, axis=0).reshape(-1)

    out_flat = _sc_edge_kernel(srcs, dsts, ts_cat, td_cat, desc)
    out2 = out_flat.reshape(ND_CAT, TSW)

    agg = {"lidar": 0.0, "radar1": 0.0, "radar2": 0.0}
    dst_types = ["lidar", "lidar", "lidar", "radar1", "radar2"]
    biases = [p_ll["bias"], p_r1l["bias"], p_r2l["bias"], p_lr1["bias"], p_lr2["bias"]]
    for t in range(5):
        nd = DST_N[t]
        seg = out2[DST_OFF[t]:DST_OFF[t] + nd]
        num = seg[:, :128]
        den = seg[:, 128:132]
        if t == 0:
            a_sl = ts_t[0][:, 128:132]
            a_dl = ts_t[0][:, 132:136]
            ex_self = jnp.exp(_leaky(a_sl + a_dl, 0.2) - t_ll)
            den = den + ex_self
            num = num + jnp.repeat(ex_self, C, axis=1) * ts_t[0][:, :128]
        outt = num / (jnp.repeat(den, C, axis=1) + 1e-16) + biases[t]
        agg[dst_types[t]] = agg[dst_types[t]] + outt
    return {nt: _leaky(v, 0.01) for nt, v in agg.items()}


def kernel(x_lidar, x_radar1, x_radar2, ei_lidar_lidar, ei_radar1_lidar,
           ei_radar2_lidar, ei_lidar_radar1, ei_lidar_radar2, params):
    srcs, dsts, desc = _prep_edges([ei_lidar_lidar, ei_radar1_lidar,
                                    ei_radar2_lidar, ei_lidar_radar1,
                                    ei_lidar_radar2])
    xs = {"lidar": _encode(x_lidar, params["enc"]["lidar"]),
          "radar1": _encode(x_radar1, params["enc"]["radar1"]),
          "radar2": _encode(x_radar2, params["enc"]["radar2"])}
    for lp in params["convs"]:
        xs = _layer(xs, lp, srcs, dsts, desc)

    lh = params["l_head"]
    res_l = _leaky(xs["lidar"] @ lh["W1"].T + lh["b1"], 0.01) @ lh["W2"].T + lh["b2"]
    mu_l = res_l[:, :2]
    sigma_l = jax.nn.softplus(res_l[:, 2:3]) + 1e-4
    rh = params["r_head"]

    def rad(nt):
        r = _leaky(xs[nt] @ rh["W1"].T + rh["b1"], 0.01) @ rh["W2"].T + rh["b2"]
        return r[:, 0:1], jax.nn.softplus(r[:, 1:2]) + 1e-4

    mu_r1, sigma_r1 = rad("radar1")
    mu_r2, sigma_r2 = rad("radar2")
    return (mu_l, sigma_l, mu_r1, sigma_r1, mu_r2, sigma_r2)
